# baseline (speedup 1.0000x reference)
import functools

import jax
import jax.numpy as jnp
from jax import lax
from jax.experimental import pallas as pl
from jax.experimental.pallas import tpu as pltpu

_VMEM_LIMIT = 56 * 1024 * 1024
_TAPS = tuple((dh, dw) for dh in (-1, 0, 1) for dw in (-1, 0, 1))


def _fused_kernel(x_ref, w_ref, mask_ref, g_ref, b_ref, o_ref,
                  rhs_ref, y1s_ref, ps_ref, ss_ref, sc_ref, sh_ref,
                  *, N, H, W, Cin, D, Cout, eps, slope):
    HW = H * W
    B = 9 * Cin
    Dh = D // 2
    c = pl.program_id(0)
    t = pl.program_id(1)

    @pl.when(t < N)
    def _conv_phase():
        for p in range(D):
            plane = x_ref[0, :, p, :].astype(jnp.bfloat16)
            for tap, (dh, dw) in enumerate(_TAPS):
                s = dh * W + dw
                if s > 0:
                    slab = jnp.concatenate(
                        [plane[:, s:], jnp.zeros((Cin, s), jnp.bfloat16)], axis=1)
                elif s < 0:
                    slab = jnp.concatenate(
                        [jnp.zeros((Cin, -s), jnp.bfloat16), plane[:, :s]], axis=1)
                else:
                    slab = plane
                if not (dh == 0 and dw == 0):
                    slab = slab * mask_ref[tap:tap + 1, :]
                rhs_ref[p * B + tap * Cin:p * B + (tap + 1) * Cin, :] = slab

        ps = jnp.zeros((Cout, 1), jnp.float32)
        ss = jnp.zeros((Cout, 1), jnp.float32)
        for d in range(D):
            qlo = max(d - 1, 0)
            qhi = min(d + 1, D - 1)
            c0 = (qlo - (d - 1)) * B
            c1 = (qhi - (d - 1) + 1) * B
            acc = jnp.dot(w_ref[:, c0:c1], rhs_ref[qlo * B:(qhi + 1) * B, :],
                          preferred_element_type=jnp.float32)
            y1s_ref[t * D + d] = acc.astype(jnp.bfloat16)
            ps = ps + jnp.sum(acc, axis=1, keepdims=True)
            ss = ss + jnp.sum(acc * acc, axis=1, keepdims=True)

        @pl.when(t == 0)
        def _():
            ps_ref[...] = ps
            ss_ref[...] = ss

        @pl.when(t > 0)
        def _():
            ps_ref[...] += ps
            ss_ref[...] += ss

    @pl.when(t == N)
    def _finalize():
        count = float(N * D * HW)
        mean = ps_ref[...] / count
        ex2 = ss_ref[...] / count
        var = jnp.maximum(ex2 - mean * mean, 0.0)
        inv_std = lax.rsqrt(var + eps)
        sc_ref[...] = g_ref[...] * inv_std
        sh_ref[...] = b_ref[...] - mean * g_ref[...] * inv_std

    @pl.when(t >= N)
    def _apply_phase():
        ta = t - N
        n_loc = c * (N // 2) + ta // 2
        half = ta % 2
        base = n_loc * D + half * Dh
        sc = sc_ref[...]
        sh = sh_ref[...]
        for g in range(Dh):
            z = y1s_ref[base + g].astype(jnp.float32) * sc + sh
            o_ref[0, :, g, :] = jnp.where(z > 0, z, slope * z)


@functools.partial(jax.jit, static_argnames=("eps", "slope"))
def _conv_bn_lrelu(x, w, gamma, beta, *, eps=1e-5, slope=0.01):
    N, Cin, D, H, W = x.shape
    Cout = w.shape[0]
    HW = H * W
    K = 27 * Cin

    x4 = x.reshape(N, Cin, D, HW)

    w_l = jnp.transpose(w, (0, 2, 3, 4, 1)).reshape(Cout, K).astype(jnp.bfloat16)

    hh = jnp.arange(H, dtype=jnp.int32).reshape(H, 1)
    ww = jnp.arange(W, dtype=jnp.int32).reshape(1, W)
    rows = []
    for dh, dw in _TAPS:
        ok = (hh + dh >= 0) & (hh + dh < H) & (ww + dw >= 0) & (ww + dw < W)
        rows.append(ok.reshape(HW))
    mask9 = jnp.stack(rows, axis=0).astype(jnp.bfloat16)

    g2 = gamma.astype(jnp.float32).reshape(Cout, 1)
    b2 = beta.astype(jnp.float32).reshape(Cout, 1)

    kern = functools.partial(_fused_kernel, N=N, H=H, W=W, Cin=Cin, D=D,
                             Cout=Cout, eps=eps, slope=slope)
    out4 = pl.pallas_call(
        kern,
        grid=(2, 2 * N),
        in_specs=[
            pl.BlockSpec((1, Cin, D, HW),
                         lambda c, t: (jnp.minimum(t, N - 1), 0, 0, 0)),
            pl.BlockSpec((Cout, K), lambda c, t: (0, 0)),
            pl.BlockSpec((9, HW), lambda c, t: (0, 0)),
            pl.BlockSpec((Cout, 1), lambda c, t: (0, 0)),
            pl.BlockSpec((Cout, 1), lambda c, t: (0, 0)),
        ],
        out_specs=pl.BlockSpec(
            (1, Cout, D // 2, HW),
            lambda c, t: (c * (N // 2) + jnp.maximum(t - N, 0) // 2, 0,
                          jnp.maximum(t - N, 0) % 2, 0)),
        out_shape=jax.ShapeDtypeStruct((N, Cout, D, HW), jnp.float32),
        scratch_shapes=[
            pltpu.VMEM((D * 9 * Cin, HW), jnp.bfloat16),
            pltpu.VMEM((N * D, Cout, HW), jnp.bfloat16),
            pltpu.VMEM((Cout, 1), jnp.float32),
            pltpu.VMEM((Cout, 1), jnp.float32),
            pltpu.VMEM((Cout, 1), jnp.float32),
            pltpu.VMEM((Cout, 1), jnp.float32),
        ],
        compiler_params=pltpu.CompilerParams(
            dimension_semantics=("parallel", "arbitrary"),
            vmem_limit_bytes=_VMEM_LIMIT),
    )(x4, w_l, mask9, g2, b2)

    return out4.reshape(N, Cout, D, H, W)


def kernel(x, w, gamma, beta):
    return _conv_bn_lrelu(x, w, gamma, beta)


# speedup vs baseline: 1.3415x; 1.3415x over previous
import functools

import jax
import jax.numpy as jnp
from jax import lax
from jax.experimental import pallas as pl
from jax.experimental.pallas import tpu as pltpu

_VMEM_LIMIT = 48 * 1024 * 1024
_TAPS = tuple((dh, dw) for dh in (-1, 0, 1) for dw in (-1, 0, 1))


def _conv_stats_kernel(x_ref, w_ref, mask_ref, y_ref, psum_ref, pssq_ref, rhs_ref,
                       *, H, W, Cin, D, Cout):
    HW = H * W
    B = 9 * Cin

    for p in range(D):
        plane = x_ref[0, :, p, :].astype(jnp.bfloat16)
        for t, (dh, dw) in enumerate(_TAPS):
            s = dh * W + dw
            if s > 0:
                slab = jnp.concatenate(
                    [plane[:, s:], jnp.zeros((Cin, s), jnp.bfloat16)], axis=1)
            elif s < 0:
                slab = jnp.concatenate(
                    [jnp.zeros((Cin, -s), jnp.bfloat16), plane[:, :s]], axis=1)
            else:
                slab = plane
            if not (dh == 0 and dw == 0):
                slab = slab * mask_ref[t:t + 1, :]
            rhs_ref[p * B + t * Cin:p * B + (t + 1) * Cin, :] = slab

    ps = jnp.zeros((Cout, 1), jnp.float32)
    ss = jnp.zeros((Cout, 1), jnp.float32)
    for d in range(D):
        qlo = max(d - 1, 0)
        qhi = min(d + 1, D - 1)
        c0 = (qlo - (d - 1)) * B
        c1 = (qhi - (d - 1) + 1) * B
        acc = jnp.dot(w_ref[:, c0:c1], rhs_ref[qlo * B:(qhi + 1) * B, :],
                      preferred_element_type=jnp.float32)
        y_ref[0, d] = acc.astype(jnp.bfloat16)
        ps = ps + jnp.sum(acc, axis=1, keepdims=True)
        ss = ss + jnp.sum(acc * acc, axis=1, keepdims=True)
    psum_ref[0] = ps
    pssq_ref[0] = ss


def _bn_lrelu_kernel(y_ref, scale_ref, shift_ref, o_ref, *, G, slope):
    for g in range(G):
        z = y_ref[0, g].astype(jnp.float32) * scale_ref[...] + shift_ref[...]
        o_ref[0, :, g, :] = jnp.where(z > 0, z, slope * z)


@functools.partial(jax.jit, static_argnames=("eps", "slope"))
def _conv_bn_lrelu(x, w, gamma, beta, *, eps=1e-5, slope=0.01):
    N, Cin, D, H, W = x.shape
    Cout = w.shape[0]
    HW = H * W
    K = 27 * Cin

    x4 = x.reshape(N, Cin, D, HW)

    w_l = jnp.transpose(w, (0, 2, 3, 4, 1)).reshape(Cout, K).astype(jnp.bfloat16)

    hh = jnp.arange(H, dtype=jnp.int32).reshape(H, 1)
    ww = jnp.arange(W, dtype=jnp.int32).reshape(1, W)
    rows = []
    for dh, dw in _TAPS:
        ok = (hh + dh >= 0) & (hh + dh < H) & (ww + dw >= 0) & (ww + dw < W)
        rows.append(ok.reshape(HW))
    mask9 = jnp.stack(rows, axis=0).astype(jnp.bfloat16)

    kern1 = functools.partial(_conv_stats_kernel, H=H, W=W, Cin=Cin, D=D, Cout=Cout)
    y1, psum, pssq = pl.pallas_call(
        kern1,
        grid=(N,),
        in_specs=[
            pl.BlockSpec((1, Cin, D, HW), lambda n: (n, 0, 0, 0)),
            pl.BlockSpec((Cout, K), lambda n: (0, 0)),
            pl.BlockSpec((9, HW), lambda n: (0, 0)),
        ],
        out_specs=[
            pl.BlockSpec((1, D, Cout, HW), lambda n: (n, 0, 0, 0)),
            pl.BlockSpec((1, Cout, 1), lambda n: (n, 0, 0)),
            pl.BlockSpec((1, Cout, 1), lambda n: (n, 0, 0)),
        ],
        out_shape=(
            jax.ShapeDtypeStruct((N, D, Cout, HW), jnp.bfloat16),
            jax.ShapeDtypeStruct((N, Cout, 1), jnp.float32),
            jax.ShapeDtypeStruct((N, Cout, 1), jnp.float32),
        ),
        scratch_shapes=[pltpu.VMEM((D * 9 * Cin, HW), jnp.bfloat16)],
        compiler_params=pltpu.CompilerParams(
            dimension_semantics=("parallel",),
            vmem_limit_bytes=_VMEM_LIMIT),
    )(x4, w_l, mask9)

    count = float(N * D * HW)
    g32 = gamma.astype(jnp.float32)
    b32 = beta.astype(jnp.float32)
    mean = jnp.sum(psum[:, :, 0], axis=0) / count
    ex2 = jnp.sum(pssq[:, :, 0], axis=0) / count
    var = jnp.maximum(ex2 - mean * mean, 0.0)
    inv_std = lax.rsqrt(var + eps)
    scale = (g32 * inv_std).reshape(Cout, 1)
    shift = (b32 - mean * g32 * inv_std).reshape(Cout, 1)

    kern2 = functools.partial(_bn_lrelu_kernel, G=D // 2, slope=slope)
    out4 = pl.pallas_call(
        kern2,
        grid=(2 * N,),
        in_specs=[
            pl.BlockSpec((1, D // 2, Cout, HW), lambda i: (i // 2, i % 2, 0, 0)),
            pl.BlockSpec((Cout, 1), lambda i: (0, 0)),
            pl.BlockSpec((Cout, 1), lambda i: (0, 0)),
        ],
        out_specs=pl.BlockSpec((1, Cout, D // 2, HW), lambda i: (i // 2, 0, i % 2, 0)),
        out_shape=jax.ShapeDtypeStruct((N, Cout, D, HW), jnp.float32),
        compiler_params=pltpu.CompilerParams(
            dimension_semantics=("parallel",),
            vmem_limit_bytes=_VMEM_LIMIT),
    )(y1, scale, shift)

    return out4.reshape(N, Cout, D, H, W)


def kernel(x, w, gamma, beta):
    return _conv_bn_lrelu(x, w, gamma, beta)


# speedup vs baseline: 1.3847x; 1.0322x over previous
import functools

import jax
import jax.numpy as jnp
from jax import lax
from jax.experimental import pallas as pl
from jax.experimental.pallas import tpu as pltpu

_VMEM_LIMIT = 48 * 1024 * 1024
_TAPS = tuple((dh, dw) for dh in (-1, 0, 1) for dw in (-1, 0, 1))


def _conv_stats_kernel(x_ref, w_ref, mask_ref, y_ref, psum_ref, pssq_ref, rhs_ref,
                       *, H, W, Cin, D, Cout):
    HW = H * W
    B = 9 * Cin

    for p in range(D):
        plane = x_ref[0, :, p, :].astype(jnp.bfloat16)
        for t, (dh, dw) in enumerate(_TAPS):
            s = dh * W + dw
            if s > 0:
                slab = jnp.concatenate(
                    [plane[:, s:], jnp.zeros((Cin, s), jnp.bfloat16)], axis=1)
            elif s < 0:
                slab = jnp.concatenate(
                    [jnp.zeros((Cin, -s), jnp.bfloat16), plane[:, :s]], axis=1)
            else:
                slab = plane
            if not (dh == 0 and dw == 0):
                slab = slab * mask_ref[t:t + 1, :]
            rhs_ref[p * B + t * Cin:p * B + (t + 1) * Cin, :] = slab

    ps = jnp.zeros((Cout, 1), jnp.float32)
    ss = jnp.zeros((Cout, 1), jnp.float32)
    for d in range(D):
        qlo = max(d - 1, 0)
        qhi = min(d + 1, D - 1)
        c0 = (qlo - (d - 1)) * B
        c1 = (qhi - (d - 1) + 1) * B
        acc = jnp.dot(w_ref[:, c0:c1], rhs_ref[qlo * B:(qhi + 1) * B, :],
                      preferred_element_type=jnp.float32)
        y_ref[0, d] = acc.astype(jnp.bfloat16)
        ps = ps + jnp.sum(acc, axis=1, keepdims=True)
        ss = ss + jnp.sum(acc * acc, axis=1, keepdims=True)
    psum_ref[0] = ps
    pssq_ref[0] = ss


def _bn_lrelu_kernel(y_ref, scale_ref, shift_ref, o_ref, *, G, slope):
    for b in range(y_ref.shape[0]):
        for g in range(G):
            z = y_ref[b, g].astype(jnp.float32) * scale_ref[...] + shift_ref[...]
            o_ref[b, :, g, :] = jnp.where(z > 0, z, slope * z)


@functools.partial(jax.jit, static_argnames=("eps", "slope"))
def _conv_bn_lrelu(x, w, gamma, beta, *, eps=1e-5, slope=0.01):
    N, Cin, D, H, W = x.shape
    Cout = w.shape[0]
    HW = H * W
    K = 27 * Cin

    x4 = x.reshape(N, Cin, D, HW)

    w_l = jnp.transpose(w, (0, 2, 3, 4, 1)).reshape(Cout, K).astype(jnp.bfloat16)

    hh = jnp.arange(H, dtype=jnp.int32).reshape(H, 1)
    ww = jnp.arange(W, dtype=jnp.int32).reshape(1, W)
    rows = []
    for dh, dw in _TAPS:
        ok = (hh + dh >= 0) & (hh + dh < H) & (ww + dw >= 0) & (ww + dw < W)
        rows.append(ok.reshape(HW))
    mask9 = jnp.stack(rows, axis=0).astype(jnp.bfloat16)

    kern1 = functools.partial(_conv_stats_kernel, H=H, W=W, Cin=Cin, D=D, Cout=Cout)
    y1, psum, pssq = pl.pallas_call(
        kern1,
        grid=(N,),
        in_specs=[
            pl.BlockSpec((1, Cin, D, HW), lambda n: (n, 0, 0, 0)),
            pl.BlockSpec((Cout, K), lambda n: (0, 0)),
            pl.BlockSpec((9, HW), lambda n: (0, 0)),
        ],
        out_specs=[
            pl.BlockSpec((1, D, Cout, HW), lambda n: (n, 0, 0, 0)),
            pl.BlockSpec((1, Cout, 1), lambda n: (n, 0, 0)),
            pl.BlockSpec((1, Cout, 1), lambda n: (n, 0, 0)),
        ],
        out_shape=(
            jax.ShapeDtypeStruct((N, D, Cout, HW), jnp.bfloat16),
            jax.ShapeDtypeStruct((N, Cout, 1), jnp.float32),
            jax.ShapeDtypeStruct((N, Cout, 1), jnp.float32),
        ),
        scratch_shapes=[pltpu.VMEM((D * 9 * Cin, HW), jnp.bfloat16)],
        compiler_params=pltpu.CompilerParams(
            dimension_semantics=("parallel",),
            vmem_limit_bytes=_VMEM_LIMIT),
    )(x4, w_l, mask9)

    count = float(N * D * HW)
    g32 = gamma.astype(jnp.float32)
    b32 = beta.astype(jnp.float32)
    mean = jnp.sum(psum[:, :, 0], axis=0) / count
    ex2 = jnp.sum(pssq[:, :, 0], axis=0) / count
    var = jnp.maximum(ex2 - mean * mean, 0.0)
    inv_std = lax.rsqrt(var + eps)
    scale = (g32 * inv_std).reshape(Cout, 1)
    shift = (b32 - mean * g32 * inv_std).reshape(Cout, 1)

    kern2 = functools.partial(_bn_lrelu_kernel, G=D, slope=slope)
    out4 = pl.pallas_call(
        kern2,
        grid=(N // 2,),
        in_specs=[
            pl.BlockSpec((2, D, Cout, HW), lambda i: (i, 0, 0, 0)),
            pl.BlockSpec((Cout, 1), lambda i: (0, 0)),
            pl.BlockSpec((Cout, 1), lambda i: (0, 0)),
        ],
        out_specs=pl.BlockSpec((2, Cout, D, HW), lambda i: (i, 0, 0, 0)),
        out_shape=jax.ShapeDtypeStruct((N, Cout, D, HW), jnp.float32),
        compiler_params=pltpu.CompilerParams(
            dimension_semantics=("parallel",),
            vmem_limit_bytes=_VMEM_LIMIT),
    )(y1, scale, shift)

    return out4.reshape(N, Cout, D, H, W)


def kernel(x, w, gamma, beta):
    return _conv_bn_lrelu(x, w, gamma, beta)


# speedup vs baseline: 1.3852x; 1.0004x over previous
import functools

import jax
import jax.numpy as jnp
from jax import lax
from jax.experimental import pallas as pl
from jax.experimental.pallas import tpu as pltpu

_VMEM_LIMIT = 48 * 1024 * 1024
_TAPS = tuple((dh, dw) for dh in (-1, 0, 1) for dw in (-1, 0, 1))


def _conv_stats_kernel(x_ref, w_ref, mask_ref, y_ref, psum_ref, pssq_ref, rhs_ref,
                       *, H, W, Cin, D, Cout):
    HW = H * W
    B = 9 * Cin

    for b in range(x_ref.shape[0]):
        for p in range(D):
            plane = x_ref[b, :, p, :].astype(jnp.bfloat16)
            for t, (dh, dw) in enumerate(_TAPS):
                s = dh * W + dw
                if s > 0:
                    slab = jnp.concatenate(
                        [plane[:, s:], jnp.zeros((Cin, s), jnp.bfloat16)], axis=1)
                elif s < 0:
                    slab = jnp.concatenate(
                        [jnp.zeros((Cin, -s), jnp.bfloat16), plane[:, :s]], axis=1)
                else:
                    slab = plane
                if not (dh == 0 and dw == 0):
                    slab = slab * mask_ref[t:t + 1, :]
                rhs_ref[p * B + t * Cin:p * B + (t + 1) * Cin, :] = slab

        ps = jnp.zeros((Cout, 1), jnp.float32)
        ss = jnp.zeros((Cout, 1), jnp.float32)
        for d in range(D):
            qlo = max(d - 1, 0)
            qhi = min(d + 1, D - 1)
            c0 = (qlo - (d - 1)) * B
            c1 = (qhi - (d - 1) + 1) * B
            acc = jnp.dot(w_ref[:, c0:c1], rhs_ref[qlo * B:(qhi + 1) * B, :],
                          preferred_element_type=jnp.float32)
            y_ref[b, d] = acc.astype(jnp.bfloat16)
            ps = ps + jnp.sum(acc, axis=1, keepdims=True)
            ss = ss + jnp.sum(acc * acc, axis=1, keepdims=True)
        psum_ref[b] = ps
        pssq_ref[b] = ss


def _bn_lrelu_kernel(y_ref, scale_ref, shift_ref, o_ref, *, G, slope):
    for b in range(y_ref.shape[0]):
        for g in range(G):
            z = y_ref[b, g].astype(jnp.float32) * scale_ref[...] + shift_ref[...]
            o_ref[b, :, g, :] = jnp.where(z > 0, z, slope * z)


@functools.partial(jax.jit, static_argnames=("eps", "slope"))
def _conv_bn_lrelu(x, w, gamma, beta, *, eps=1e-5, slope=0.01):
    N, Cin, D, H, W = x.shape
    Cout = w.shape[0]
    HW = H * W
    K = 27 * Cin

    x4 = x.reshape(N, Cin, D, HW)

    w_l = jnp.transpose(w, (0, 2, 3, 4, 1)).reshape(Cout, K).astype(jnp.bfloat16)

    hh = jnp.arange(H, dtype=jnp.int32).reshape(H, 1)
    ww = jnp.arange(W, dtype=jnp.int32).reshape(1, W)
    rows = []
    for dh, dw in _TAPS:
        ok = (hh + dh >= 0) & (hh + dh < H) & (ww + dw >= 0) & (ww + dw < W)
        rows.append(ok.reshape(HW))
    mask9 = jnp.stack(rows, axis=0).astype(jnp.bfloat16)

    kern1 = functools.partial(_conv_stats_kernel, H=H, W=W, Cin=Cin, D=D, Cout=Cout)
    y1, psum, pssq = pl.pallas_call(
        kern1,
        grid=(N // 2,),
        in_specs=[
            pl.BlockSpec((2, Cin, D, HW), lambda n: (n, 0, 0, 0)),
            pl.BlockSpec((Cout, K), lambda n: (0, 0)),
            pl.BlockSpec((9, HW), lambda n: (0, 0)),
        ],
        out_specs=[
            pl.BlockSpec((2, D, Cout, HW), lambda n: (n, 0, 0, 0)),
            pl.BlockSpec((2, Cout, 1), lambda n: (n, 0, 0)),
            pl.BlockSpec((2, Cout, 1), lambda n: (n, 0, 0)),
        ],
        out_shape=(
            jax.ShapeDtypeStruct((N, D, Cout, HW), jnp.bfloat16),
            jax.ShapeDtypeStruct((N, Cout, 1), jnp.float32),
            jax.ShapeDtypeStruct((N, Cout, 1), jnp.float32),
        ),
        scratch_shapes=[pltpu.VMEM((D * 9 * Cin, HW), jnp.bfloat16)],
        compiler_params=pltpu.CompilerParams(
            dimension_semantics=("parallel",),
            vmem_limit_bytes=_VMEM_LIMIT),
    )(x4, w_l, mask9)

    count = float(N * D * HW)
    g32 = gamma.astype(jnp.float32)
    b32 = beta.astype(jnp.float32)
    mean = jnp.sum(psum[:, :, 0], axis=0) / count
    ex2 = jnp.sum(pssq[:, :, 0], axis=0) / count
    var = jnp.maximum(ex2 - mean * mean, 0.0)
    inv_std = lax.rsqrt(var + eps)
    scale = (g32 * inv_std).reshape(Cout, 1)
    shift = (b32 - mean * g32 * inv_std).reshape(Cout, 1)

    kern2 = functools.partial(_bn_lrelu_kernel, G=D, slope=slope)
    out4 = pl.pallas_call(
        kern2,
        grid=(N // 2,),
        in_specs=[
            pl.BlockSpec((2, D, Cout, HW), lambda i: (i, 0, 0, 0)),
            pl.BlockSpec((Cout, 1), lambda i: (0, 0)),
            pl.BlockSpec((Cout, 1), lambda i: (0, 0)),
        ],
        out_specs=pl.BlockSpec((2, Cout, D, HW), lambda i: (i, 0, 0, 0)),
        out_shape=jax.ShapeDtypeStruct((N, Cout, D, HW), jnp.float32),
        compiler_params=pltpu.CompilerParams(
            dimension_semantics=("parallel",),
            vmem_limit_bytes=_VMEM_LIMIT),
    )(y1, scale, shift)

    return out4.reshape(N, Cout, D, H, W)


def kernel(x, w, gamma, beta):
    return _conv_bn_lrelu(x, w, gamma, beta)
